# baseline (speedup 1.0000x reference)
import functools

import jax
import jax.numpy as jnp
from jax.experimental import pallas as pl
from jax.experimental.pallas import tpu as pltpu

_WEIGHTS = (0.33, 0.33, 0.34)
_INITIAL_RADIUS = 0.05


def _radii(base):
    return (base * 5.0, base * 10.0, base * 20.0)


_R_L0 = _radii(_INITIAL_RADIUS)
_R_L1 = _radii(_INITIAL_RADIUS * 4)
_R_L2 = _radii(_INITIAL_RADIUS * 16)
_R_L3 = _radii(_INITIAL_RADIUS * 32)


def _gauss_agg(q, sp, fea, radii):
    qq = jnp.sum(q * q, axis=1, keepdims=True)
    ss = jnp.sum(sp * sp, axis=1, keepdims=True)
    qs = jnp.dot(q, sp.T, preferred_element_type=jnp.float32)
    d2 = qq + ss.T - 2.0 * qs
    faug = jnp.concatenate(
        [fea, jnp.ones((fea.shape[0], 1), jnp.float32)], axis=1)
    h = None
    for r, w in zip(radii, _WEIGHTS):
        g = jnp.exp(d2 * (-1.0 / (r * r)))
        nd = jnp.dot(g, faug, preferred_element_type=jnp.float32)
        term = nd[:, :-1] * (w / (nd[:, -1:] + 1e-8))
        h = term if h is None else h + term
    return h


def _mlp_apply(h, wrefs, wi, n_post, relu_mask, skip=None):
    for li in range(n_post):
        if li == 0 and skip is not None:
            Wa = wrefs[wi][...]
            Wb = wrefs[wi + 1][...]
            b = wrefs[wi + 2][...]
            wi += 3
            h = (jnp.dot(h, Wa, preferred_element_type=jnp.float32)
                 + jnp.dot(skip, Wb, preferred_element_type=jnp.float32)
                 + b)
        else:
            W = wrefs[wi][...]
            b = wrefs[wi + 1][...]
            wi += 2
            h = jnp.dot(h, W, preferred_element_type=jnp.float32) + b
        if relu_mask[li]:
            h = jax.nn.relu(h)
    return h, wi


def _sconv_body(radii, n_pre, has_skip, n_post, relu_mask, q_ref, s_ref,
                f_ref, *rest):
    out_ref = rest[-1]
    rest = rest[:-1]
    skip_ref = None
    if has_skip:
        skip_ref = rest[0]
        rest = rest[1:]
    wrefs = rest

    q = q_ref[0]
    sp = s_ref[0]
    fea = f_ref[0]

    wi = 0
    for _ in range(n_pre):
        W = wrefs[wi][...]
        b = wrefs[wi + 1][...]
        wi += 2
        fea = jax.nn.relu(
            jnp.dot(fea, W, preferred_element_type=jnp.float32) + b)

    h = _gauss_agg(q, sp, fea, radii)
    h, wi = _mlp_apply(h, wrefs, wi, n_post, relu_mask,
                       skip=None if skip_ref is None else skip_ref[0])
    out_ref[0] = h


def _sconv_pair_body(radii, has_skip, n_post1, relu1, n_post2, relu2,
                     q_ref, s_ref, f_ref, *rest):
    out_ref = rest[-1]
    rest = rest[:-1]
    skip_ref = None
    if has_skip:
        skip_ref = rest[0]
        rest = rest[1:]
    wrefs = rest

    q = q_ref[0]
    sp = s_ref[0]
    fea = f_ref[0]

    h = _gauss_agg(q, sp, fea, radii)
    h, wi = _mlp_apply(h, wrefs, 0, n_post1, relu1,
                       skip=None if skip_ref is None else skip_ref[0])
    h = _gauss_agg(q, q, h, radii)
    h, wi = _mlp_apply(h, wrefs, wi, n_post2, relu2)
    out_ref[0] = h


def _sconv_call(q_pc, s_pc, s_fea, radii, pre=(), post=(), skip=None,
                q_block=512):
    B, Q, _ = q_pc.shape
    S = s_pc.shape[1]
    Qb = min(Q, q_block)
    grid = (B, Q // Qb)

    cf = s_fea.shape[-1]
    for (W, _b) in pre:
        cf = W.shape[1]
    c_out = cf
    relu_mask = []
    for (W, _b, act) in post:
        c_out = W.shape[1]
        relu_mask.append(act)

    operands = [q_pc, s_pc, s_fea]
    in_specs = [
        pl.BlockSpec((1, Qb, 3), lambda b, qi: (b, qi, 0)),
        pl.BlockSpec((1, S, 3), lambda b, qi: (b, 0, 0)),
        pl.BlockSpec((1, S, s_fea.shape[-1]), lambda b, qi: (b, 0, 0)),
    ]
    if skip is not None:
        operands.append(skip)
        in_specs.append(
            pl.BlockSpec((1, Qb, skip.shape[-1]), lambda b, qi: (b, qi, 0)))

    def _add_w(W, b2d):
        operands.append(W)
        operands.append(b2d)
        in_specs.append(pl.BlockSpec(W.shape, lambda b, qi: (0, 0)))
        in_specs.append(pl.BlockSpec(b2d.shape, lambda b, qi: (0, 0)))

    for (W, b) in pre:
        _add_w(W, b.reshape(1, -1))
    for li, (W, b, _act) in enumerate(post):
        if li == 0 and skip is not None:
            cagg = cf
            Wa, Wb = W[:cagg], W[cagg:]
            operands.extend([Wa, Wb, b.reshape(1, -1)])
            in_specs.append(pl.BlockSpec(Wa.shape, lambda b, qi: (0, 0)))
            in_specs.append(pl.BlockSpec(Wb.shape, lambda b, qi: (0, 0)))
            in_specs.append(
                pl.BlockSpec((1, W.shape[1]), lambda b, qi: (0, 0)))
        else:
            _add_w(W, b.reshape(1, -1))

    body = functools.partial(_sconv_body, radii, len(pre), skip is not None,
                             len(post), tuple(relu_mask))
    return pl.pallas_call(
        body,
        grid=grid,
        in_specs=in_specs,
        out_specs=pl.BlockSpec((1, Qb, c_out), lambda b, qi: (b, qi, 0)),
        out_shape=jax.ShapeDtypeStruct((B, Q, c_out), jnp.float32),
        compiler_params=pltpu.CompilerParams(
            dimension_semantics=("arbitrary", "arbitrary")),
    )(*operands)


def _sconv_pair_call(q_pc, s_pc, s_fea, radii, post1, post2, skip=None):
    B, Q, _ = q_pc.shape
    S = s_pc.shape[1]

    relu1 = []
    c1 = s_fea.shape[-1]
    for (W, _b, act) in post1:
        c1 = W.shape[1]
        relu1.append(act)
    relu2 = []
    c_out = c1
    for (W, _b, act) in post2:
        c_out = W.shape[1]
        relu2.append(act)

    operands = [q_pc, s_pc, s_fea]
    in_specs = [
        pl.BlockSpec((1, Q, 3), lambda b: (b, 0, 0)),
        pl.BlockSpec((1, S, 3), lambda b: (b, 0, 0)),
        pl.BlockSpec((1, S, s_fea.shape[-1]), lambda b: (b, 0, 0)),
    ]
    if skip is not None:
        operands.append(skip)
        in_specs.append(
            pl.BlockSpec((1, Q, skip.shape[-1]), lambda b: (b, 0, 0)))

    def _add_w(W, b2d):
        operands.append(W)
        operands.append(b2d)
        in_specs.append(pl.BlockSpec(W.shape, lambda b: (0, 0)))
        in_specs.append(pl.BlockSpec(b2d.shape, lambda b: (0, 0)))

    for li, (W, b, _act) in enumerate(post1):
        if li == 0 and skip is not None:
            cagg = s_fea.shape[-1]
            Wa, Wb = W[:cagg], W[cagg:]
            operands.extend([Wa, Wb, b.reshape(1, -1)])
            in_specs.append(pl.BlockSpec(Wa.shape, lambda b: (0, 0)))
            in_specs.append(pl.BlockSpec(Wb.shape, lambda b: (0, 0)))
            in_specs.append(pl.BlockSpec((1, W.shape[1]), lambda b: (0, 0)))
        else:
            _add_w(W, b.reshape(1, -1))
    for (W, b, _act) in post2:
        _add_w(W, b.reshape(1, -1))

    body = functools.partial(_sconv_pair_body, radii, skip is not None,
                             len(post1), tuple(relu1),
                             len(post2), tuple(relu2))
    return pl.pallas_call(
        body,
        grid=(B,),
        in_specs=in_specs,
        out_specs=pl.BlockSpec((1, Q, c_out), lambda b: (b, 0, 0)),
        out_shape=jax.ShapeDtypeStruct((B, Q, c_out), jnp.float32),
        compiler_params=pltpu.CompilerParams(
            dimension_semantics=("arbitrary",)),
    )(*operands)


def _fps_body(npoints, N, xs_ref, ys_ref, zs_ref, ox_ref, oy_ref, oz_ref):
    xs = xs_ref[...]
    ys = ys_ref[...]
    zs = zs_ref[...]
    Bb, SUB, _ = xs.shape
    lin = (jax.lax.broadcasted_iota(jnp.int32, (1, SUB, 128), 1) * 128
           + jax.lax.broadcasted_iota(jnp.int32, (1, SUB, 128), 2))

    def step(i, carry):
        dist, last = carry
        oh = (lin == last).astype(jnp.float32)
        lx = jnp.sum(xs * oh, axis=(1, 2), keepdims=True)
        ly = jnp.sum(ys * oh, axis=(1, 2), keepdims=True)
        lz = jnp.sum(zs * oh, axis=(1, 2), keepdims=True)
        ox_ref[0, pl.ds(i, 1), :] = lx.reshape(1, Bb)
        oy_ref[0, pl.ds(i, 1), :] = ly.reshape(1, Bb)
        oz_ref[0, pl.ds(i, 1), :] = lz.reshape(1, Bb)
        d = (xs - lx) ** 2 + (ys - ly) ** 2 + (zs - lz) ** 2
        dist = jnp.minimum(dist, d)
        m = jnp.max(dist, axis=(1, 2), keepdims=True)
        nxt = jnp.min(jnp.where(dist == m, lin, N), axis=(1, 2),
                      keepdims=True)
        return dist, nxt

    jax.lax.fori_loop(
        0, npoints, step,
        (jnp.full((Bb, SUB, 128), 1e10, jnp.float32),
         jnp.zeros((Bb, 1, 1), jnp.int32)))


def _fps_call(pts, npoints):
    B, N, _ = pts.shape
    sub = N // 128
    xs = pts[:, :, 0].reshape(B, sub, 128)
    ys = pts[:, :, 1].reshape(B, sub, 128)
    zs = pts[:, :, 2].reshape(B, sub, 128)
    Bb = B
    ng = B // Bb
    out_sd = jax.ShapeDtypeStruct((ng, npoints, Bb), jnp.float32)
    in_spec = pl.BlockSpec((Bb, sub, 128), lambda i: (i, 0, 0))
    out_spec = pl.BlockSpec((1, npoints, Bb), lambda i: (i, 0, 0))
    ox, oy, oz = pl.pallas_call(
        functools.partial(_fps_body, npoints, N),
        grid=(ng,),
        in_specs=[in_spec, in_spec, in_spec],
        out_specs=(out_spec, out_spec, out_spec),
        out_shape=(out_sd, out_sd, out_sd),
        compiler_params=pltpu.CompilerParams(
            dimension_semantics=("arbitrary",)),
    )(xs, ys, zs)
    def _fix(o):
        return o.transpose(0, 2, 1).reshape(B, npoints)
    return jnp.stack([_fix(ox), _fix(oy), _fix(oz)], axis=-1)


def kernel(pc1, feature1, params):
    p = params

    def wb(name, act=None):
        if act is None:
            return (p[name + "_W"], p[name + "_b"])
        return (p[name + "_W"], p[name + "_b"], act)

    l0 = pc1
    f0 = _sconv_call(l0, l0, feature1, _R_L0,
                     pre=(wb("cc0_0"), wb("cc0_1")),
                     post=(wb("cc0_2", True),))
    l1 = _fps_call(l0, 512)
    f1 = _sconv_pair_call(l1, l0, f0, _R_L1,
                          post1=(wb("cc1_0", True), wb("cc1_1", True)),
                          post2=(wb("cc1_2", True),))
    l2 = _fps_call(l1, 128)
    f2 = _sconv_pair_call(l2, l1, f1, _R_L2,
                          post1=(wb("cc2_0", True), wb("cc2_1", True)),
                          post2=(wb("cc2_2", True),))
    l3 = _fps_call(l2, 64)
    f3 = _sconv_pair_call(l3, l2, f2, _R_L3,
                          post1=(wb("cc3_0", True), wb("cc3_1", True)),
                          post2=(wb("cc3_2", True),))
    f2 = _sconv_pair_call(l2, l3, f3, _R_L2,
                          post1=(wb("cc2_3", True), wb("cc2_4", True)),
                          post2=(wb("cc2_5", True),), skip=f2)
    f1 = _sconv_pair_call(l1, l2, f2, _R_L1,
                          post1=(wb("cc1_3", True), wb("cc1_4", True)),
                          post2=(wb("cc1_5", True),), skip=f1)
    f0 = _sconv_call(l0, l1, f1, _R_L0,
                     post=(wb("cc0_3", True), wb("cc0_4", True)), skip=f0)
    out = _sconv_call(l0, l0, f0, _R_L0,
                      post=(wb("cc0_5", True), wb("refine", False)))
    return out


# speedup vs baseline: 1.0726x; 1.0726x over previous
import functools

import jax
import jax.numpy as jnp
from jax.experimental import pallas as pl
from jax.experimental.pallas import tpu as pltpu

_WEIGHTS = (0.33, 0.33, 0.34)
_INITIAL_RADIUS = 0.05


def _radii(base):
    return (base * 5.0, base * 10.0, base * 20.0)


_R_L0 = _radii(_INITIAL_RADIUS)
_R_L1 = _radii(_INITIAL_RADIUS * 4)
_R_L2 = _radii(_INITIAL_RADIUS * 16)
_R_L3 = _radii(_INITIAL_RADIUS * 32)


def _gauss_agg(q, sp, fea, radii):
    qq = jnp.sum(q * q, axis=1, keepdims=True)
    ss = jnp.sum(sp * sp, axis=1, keepdims=True)
    qs = jnp.dot(q, sp.T, preferred_element_type=jnp.float32)
    d2 = qq + ss.T - 2.0 * qs
    faug = jnp.concatenate(
        [fea, jnp.ones((fea.shape[0], 1), jnp.float32)], axis=1)
    h = None
    for r, w in zip(radii, _WEIGHTS):
        g = jnp.exp(d2 * (-1.0 / (r * r)))
        nd = jnp.dot(g, faug, preferred_element_type=jnp.float32)
        term = nd[:, :-1] * (w / (nd[:, -1:] + 1e-8))
        h = term if h is None else h + term
    return h


def _mlp_apply(h, wrefs, wi, n_post, relu_mask, skip=None):
    for li in range(n_post):
        if li == 0 and skip is not None:
            Wa = wrefs[wi][...]
            Wb = wrefs[wi + 1][...]
            b = wrefs[wi + 2][...]
            wi += 3
            h = (jnp.dot(h, Wa, preferred_element_type=jnp.float32)
                 + jnp.dot(skip, Wb, preferred_element_type=jnp.float32)
                 + b)
        else:
            W = wrefs[wi][...]
            b = wrefs[wi + 1][...]
            wi += 2
            h = jnp.dot(h, W, preferred_element_type=jnp.float32) + b
        if relu_mask[li]:
            h = jax.nn.relu(h)
    return h, wi


def _sconv_body(radii, n_pre, has_skip, n_post, relu_mask, q_ref, s_ref,
                f_ref, *rest):
    out_ref = rest[-1]
    rest = rest[:-1]
    skip_ref = None
    if has_skip:
        skip_ref = rest[0]
        rest = rest[1:]
    wrefs = rest

    q = q_ref[0]
    sp = s_ref[0]
    fea = f_ref[0]

    wi = 0
    for _ in range(n_pre):
        W = wrefs[wi][...]
        b = wrefs[wi + 1][...]
        wi += 2
        fea = jax.nn.relu(
            jnp.dot(fea, W, preferred_element_type=jnp.float32) + b)

    h = _gauss_agg(q, sp, fea, radii)
    h, wi = _mlp_apply(h, wrefs, wi, n_post, relu_mask,
                       skip=None if skip_ref is None else skip_ref[0])
    out_ref[0] = h


def _sconv_pair_body(radii, has_skip, n_post1, relu1, n_post2, relu2,
                     q_ref, s_ref, f_ref, *rest):
    out_ref = rest[-1]
    rest = rest[:-1]
    skip_ref = None
    if has_skip:
        skip_ref = rest[0]
        rest = rest[1:]
    wrefs = rest

    q = q_ref[0]
    sp = s_ref[0]
    fea = f_ref[0]

    h = _gauss_agg(q, sp, fea, radii)
    h, wi = _mlp_apply(h, wrefs, 0, n_post1, relu1,
                       skip=None if skip_ref is None else skip_ref[0])
    h = _gauss_agg(q, q, h, radii)
    h, wi = _mlp_apply(h, wrefs, wi, n_post2, relu2)
    out_ref[0] = h


def _sconv_call(q_pc, s_pc, s_fea, radii, pre=(), post=(), skip=None,
                q_block=1024):
    B, Q, _ = q_pc.shape
    S = s_pc.shape[1]
    Qb = min(Q, q_block)
    grid = (B, Q // Qb)

    cf = s_fea.shape[-1]
    for (W, _b) in pre:
        cf = W.shape[1]
    c_out = cf
    relu_mask = []
    for (W, _b, act) in post:
        c_out = W.shape[1]
        relu_mask.append(act)

    operands = [q_pc, s_pc, s_fea]
    in_specs = [
        pl.BlockSpec((1, Qb, 3), lambda b, qi: (b, qi, 0)),
        pl.BlockSpec((1, S, 3), lambda b, qi: (b, 0, 0)),
        pl.BlockSpec((1, S, s_fea.shape[-1]), lambda b, qi: (b, 0, 0)),
    ]
    if skip is not None:
        operands.append(skip)
        in_specs.append(
            pl.BlockSpec((1, Qb, skip.shape[-1]), lambda b, qi: (b, qi, 0)))

    def _add_w(W, b2d):
        operands.append(W)
        operands.append(b2d)
        in_specs.append(pl.BlockSpec(W.shape, lambda b, qi: (0, 0)))
        in_specs.append(pl.BlockSpec(b2d.shape, lambda b, qi: (0, 0)))

    for (W, b) in pre:
        _add_w(W, b.reshape(1, -1))
    for li, (W, b, _act) in enumerate(post):
        if li == 0 and skip is not None:
            cagg = cf
            Wa, Wb = W[:cagg], W[cagg:]
            operands.extend([Wa, Wb, b.reshape(1, -1)])
            in_specs.append(pl.BlockSpec(Wa.shape, lambda b, qi: (0, 0)))
            in_specs.append(pl.BlockSpec(Wb.shape, lambda b, qi: (0, 0)))
            in_specs.append(
                pl.BlockSpec((1, W.shape[1]), lambda b, qi: (0, 0)))
        else:
            _add_w(W, b.reshape(1, -1))

    body = functools.partial(_sconv_body, radii, len(pre), skip is not None,
                             len(post), tuple(relu_mask))
    return pl.pallas_call(
        body,
        grid=grid,
        in_specs=in_specs,
        out_specs=pl.BlockSpec((1, Qb, c_out), lambda b, qi: (b, qi, 0)),
        out_shape=jax.ShapeDtypeStruct((B, Q, c_out), jnp.float32),
        compiler_params=pltpu.CompilerParams(
            dimension_semantics=("arbitrary", "arbitrary")),
    )(*operands)


def _sconv_pair_call(q_pc, s_pc, s_fea, radii, post1, post2, skip=None):
    B, Q, _ = q_pc.shape
    S = s_pc.shape[1]

    relu1 = []
    c1 = s_fea.shape[-1]
    for (W, _b, act) in post1:
        c1 = W.shape[1]
        relu1.append(act)
    relu2 = []
    c_out = c1
    for (W, _b, act) in post2:
        c_out = W.shape[1]
        relu2.append(act)

    operands = [q_pc, s_pc, s_fea]
    in_specs = [
        pl.BlockSpec((1, Q, 3), lambda b: (b, 0, 0)),
        pl.BlockSpec((1, S, 3), lambda b: (b, 0, 0)),
        pl.BlockSpec((1, S, s_fea.shape[-1]), lambda b: (b, 0, 0)),
    ]
    if skip is not None:
        operands.append(skip)
        in_specs.append(
            pl.BlockSpec((1, Q, skip.shape[-1]), lambda b: (b, 0, 0)))

    def _add_w(W, b2d):
        operands.append(W)
        operands.append(b2d)
        in_specs.append(pl.BlockSpec(W.shape, lambda b: (0, 0)))
        in_specs.append(pl.BlockSpec(b2d.shape, lambda b: (0, 0)))

    for li, (W, b, _act) in enumerate(post1):
        if li == 0 and skip is not None:
            cagg = s_fea.shape[-1]
            Wa, Wb = W[:cagg], W[cagg:]
            operands.extend([Wa, Wb, b.reshape(1, -1)])
            in_specs.append(pl.BlockSpec(Wa.shape, lambda b: (0, 0)))
            in_specs.append(pl.BlockSpec(Wb.shape, lambda b: (0, 0)))
            in_specs.append(pl.BlockSpec((1, W.shape[1]), lambda b: (0, 0)))
        else:
            _add_w(W, b.reshape(1, -1))
    for (W, b, _act) in post2:
        _add_w(W, b.reshape(1, -1))

    body = functools.partial(_sconv_pair_body, radii, skip is not None,
                             len(post1), tuple(relu1),
                             len(post2), tuple(relu2))
    return pl.pallas_call(
        body,
        grid=(B,),
        in_specs=in_specs,
        out_specs=pl.BlockSpec((1, Q, c_out), lambda b: (b, 0, 0)),
        out_shape=jax.ShapeDtypeStruct((B, Q, c_out), jnp.float32),
        compiler_params=pltpu.CompilerParams(
            dimension_semantics=("arbitrary",)),
    )(*operands)


def _fps_body(npoints, N, xs_ref, ys_ref, zs_ref, ox_ref, oy_ref, oz_ref):
    xs = xs_ref[...]
    ys = ys_ref[...]
    zs = zs_ref[...]
    Bb, SUB, _ = xs.shape
    lin = (jax.lax.broadcasted_iota(jnp.int32, (1, SUB, 128), 1) * 128
           + jax.lax.broadcasted_iota(jnp.int32, (1, SUB, 128), 2))

    def step(i, carry):
        dist, last = carry
        oh = (lin == last).astype(jnp.float32)
        lx = jnp.sum(xs * oh, axis=(1, 2), keepdims=True)
        ly = jnp.sum(ys * oh, axis=(1, 2), keepdims=True)
        lz = jnp.sum(zs * oh, axis=(1, 2), keepdims=True)
        ox_ref[0, pl.ds(i, 1), :] = lx.reshape(1, Bb)
        oy_ref[0, pl.ds(i, 1), :] = ly.reshape(1, Bb)
        oz_ref[0, pl.ds(i, 1), :] = lz.reshape(1, Bb)
        d = (xs - lx) ** 2 + (ys - ly) ** 2 + (zs - lz) ** 2
        dist = jnp.minimum(dist, d)
        m = jnp.max(dist, axis=(1, 2), keepdims=True)
        nxt = jnp.min(jnp.where(dist == m, lin, N), axis=(1, 2),
                      keepdims=True)
        return dist, nxt

    jax.lax.fori_loop(
        0, npoints, step,
        (jnp.full((Bb, SUB, 128), 1e10, jnp.float32),
         jnp.zeros((Bb, 1, 1), jnp.int32)))


def _fps_call(pts, npoints):
    B, N, _ = pts.shape
    sub = N // 128
    xs = pts[:, :, 0].reshape(B, sub, 128)
    ys = pts[:, :, 1].reshape(B, sub, 128)
    zs = pts[:, :, 2].reshape(B, sub, 128)
    Bb = B
    ng = B // Bb
    out_sd = jax.ShapeDtypeStruct((ng, npoints, Bb), jnp.float32)
    in_spec = pl.BlockSpec((Bb, sub, 128), lambda i: (i, 0, 0))
    out_spec = pl.BlockSpec((1, npoints, Bb), lambda i: (i, 0, 0))
    ox, oy, oz = pl.pallas_call(
        functools.partial(_fps_body, npoints, N),
        grid=(ng,),
        in_specs=[in_spec, in_spec, in_spec],
        out_specs=(out_spec, out_spec, out_spec),
        out_shape=(out_sd, out_sd, out_sd),
        compiler_params=pltpu.CompilerParams(
            dimension_semantics=("arbitrary",)),
    )(xs, ys, zs)
    def _fix(o):
        return o.transpose(0, 2, 1).reshape(B, npoints)
    return jnp.stack([_fix(ox), _fix(oy), _fix(oz)], axis=-1)


def kernel(pc1, feature1, params):
    p = params

    def wb(name, act=None):
        if act is None:
            return (p[name + "_W"], p[name + "_b"])
        return (p[name + "_W"], p[name + "_b"], act)

    l0 = pc1
    f0 = _sconv_call(l0, l0, feature1, _R_L0,
                     pre=(wb("cc0_0"), wb("cc0_1")),
                     post=(wb("cc0_2", True),))
    l1 = _fps_call(l0, 512)
    f1 = _sconv_pair_call(l1, l0, f0, _R_L1,
                          post1=(wb("cc1_0", True), wb("cc1_1", True)),
                          post2=(wb("cc1_2", True),))
    l2 = _fps_call(l1, 128)
    f2 = _sconv_pair_call(l2, l1, f1, _R_L2,
                          post1=(wb("cc2_0", True), wb("cc2_1", True)),
                          post2=(wb("cc2_2", True),))
    l3 = _fps_call(l2, 64)
    f3 = _sconv_pair_call(l3, l2, f2, _R_L3,
                          post1=(wb("cc3_0", True), wb("cc3_1", True)),
                          post2=(wb("cc3_2", True),))
    f2 = _sconv_pair_call(l2, l3, f3, _R_L2,
                          post1=(wb("cc2_3", True), wb("cc2_4", True)),
                          post2=(wb("cc2_5", True),), skip=f2)
    f1 = _sconv_pair_call(l1, l2, f2, _R_L1,
                          post1=(wb("cc1_3", True), wb("cc1_4", True)),
                          post2=(wb("cc1_5", True),), skip=f1)
    f0 = _sconv_call(l0, l1, f1, _R_L0,
                     post=(wb("cc0_3", True), wb("cc0_4", True)), skip=f0)
    out = _sconv_call(l0, l0, f0, _R_L0,
                      post=(wb("cc0_5", True), wb("refine", False)))
    return out
